# baseline (speedup 1.0000x reference)
import functools

import jax
import jax.numpy as jnp
from jax import lax
from jax.experimental import pallas as pl
from jax.experimental.pallas import tpu as pltpu
from jax.experimental.pallas import tpu_sc as plsc

B, S, D, V = 4096, 200, 64, 1000000
NC, NS = 2, 16
NW = NC * NS
BT = B // 128
FT = D // 8
LANES = 16
STRIDE = 129

NT_FULL = V // 128
TAIL = V - NT_FULL * 128
PER_W_BASE = NT_FULL // NW
PER_W_EXTRA = NT_FULL % NW

_mesh = plsc.VectorSubcoreMesh(core_axis_name="c", subcore_axis_name="s")


@functools.partial(
    pl.kernel,
    out_type=jax.ShapeDtypeStruct((V, 128), jnp.float32),
    mesh=_mesh,
    scratch_types=[
        pltpu.VMEM((2, D, 128), jnp.float32),
        pltpu.VMEM((2, 128, STRIDE), jnp.float32),
        pltpu.SemaphoreType.DMA,
        pltpu.SemaphoreType.DMA,
        pltpu.SemaphoreType.DMA,
        pltpu.SemaphoreType.DMA,
    ],
    compiler_params=pltpu.CompilerParams(use_tc_tiling_on_sc=True,
                                         needs_layout_passes=False),
)
def _prep(tablet_hbm, tail_hbm, scr_hbm, in_v, st_v, gsem0, gsem1, wsem0, wsem1):
    w = lax.axis_index("s") * NC + lax.axis_index("c")
    gsems = (gsem0, gsem1)
    wsems = (wsem0, wsem1)
    base = w * PER_W_BASE + jnp.minimum(w, PER_W_EXTRA)
    cnt = PER_W_BASE + jnp.where(w < PER_W_EXTRA, 1, 0)

    def rt_of(j):
        return pl.multiple_of((base + j) * 128, 128)

    def fire_read(buf, j):
        pltpu.async_copy(tablet_hbm.at[:, pl.ds(rt_of(j), 128)], in_v.at[buf],
                         gsems[buf])

    def wait_read(buf, j):
        pltpu.make_async_copy(tablet_hbm.at[:, pl.ds(rt_of(j), 128)],
                              in_v.at[buf], gsems[buf]).wait()

    def fire_write(buf, j):
        pltpu.async_copy(st_v.at[buf, :, pl.ds(0, 128)],
                         scr_hbm.at[pl.ds(rt_of(j), 128)], wsems[buf])

    def wait_write(buf, j):
        pltpu.make_async_copy(st_v.at[buf, :, pl.ds(0, 128)],
                              scr_hbm.at[pl.ds(rt_of(j), 128)],
                              wsems[buf]).wait()

    lane_iota = lax.iota(jnp.int32, LANES)
    tvs = [jnp.int32(t0 * LANES) + lane_iota for t0 in range(128 // LANES)]

    def transpose(buf):
        @plsc.parallel_loop(0, D, unroll=4)
        def _f(f):
            fv = jnp.full((LANES,), f, jnp.int32)
            for t0 in range(128 // LANES):
                v = in_v[buf, f, pl.ds(t0 * LANES, LANES)]
                plsc.store_scatter(st_v.at[buf], [tvs[t0], fv], v)

    @pl.when(cnt > 0)
    def _():
        fire_read(0, 0)

    @pl.when(cnt > 1)
    def _():
        fire_read(1, 1)

    def body(j, _):
        for buf in (0, 1):
            jj = 2 * j + buf

            @pl.when(jj < cnt)
            def _():
                wait_read(buf, jj)

                @pl.when(jj >= 2)
                def _():
                    wait_write(buf, jj - 2)

                transpose(buf)
                fire_write(buf, jj)

                @pl.when(jj + 2 < cnt)
                def _():
                    fire_read(buf, jj + 2)

        return 0

    lax.fori_loop(0, (PER_W_BASE + 2) // 2, body, 0)

    @pl.when(cnt > 1)
    def _():
        wait_write(0, cnt - 2)

    @pl.when(cnt > 0)
    def _():
        wait_write(1, cnt - 1)

    @pl.when(w == NW - 1)
    def _():
        pltpu.sync_copy(tail_hbm, in_v.at[0, :, :])
        pltpu.sync_copy(in_v.at[0, :, :],
                        scr_hbm.at[pl.ds(NT_FULL * 128, TAIL)])


@functools.partial(
    pl.kernel,
    out_type=jax.ShapeDtypeStruct((S, FT, BT, 8, 128), jnp.float32),
    mesh=_mesh,
    scratch_types=[
        pltpu.VMEM((S, 128), jnp.int32),
        pltpu.VMEM((S, D), jnp.float32),
        pltpu.VMEM((2, 128, 128), jnp.float32),
        pltpu.VMEM((2, FT, 8, STRIDE), jnp.float32),
        pltpu.SemaphoreType.DMA,
        pltpu.SemaphoreType.DMA,
        pltpu.SemaphoreType.DMA,
        pltpu.SemaphoreType.DMA,
    ],
    compiler_params=pltpu.CompilerParams(use_tc_tiling_on_sc=False,
                                         needs_layout_passes=False),
)
def _emb(idxt_hbm, scr_hbm, pos_hbm, out_hbm,
         idx_v, pos_v, rows_v, outt_v, gsem0, gsem1, wsem0, wsem1):
    w = lax.axis_index("s") * NC + lax.axis_index("c")
    gsems = (gsem0, gsem1)
    wsems = (wsem0, wsem1)

    pltpu.sync_copy(idxt_hbm.at[:, pl.ds(pl.multiple_of(w * 128, 128), 128)], idx_v)
    pltpu.sync_copy(pos_hbm, pos_v)

    def fire_gather(buf, s):
        pltpu.async_copy(scr_hbm.at[idx_v.at[s]], rows_v.at[buf], gsems[buf])

    def wait_gather(buf, s):
        pltpu.make_async_copy(scr_hbm.at[idx_v.at[s]], rows_v.at[buf],
                              gsems[buf]).wait()

    def fire_write(buf, s):
        pltpu.async_copy(outt_v.at[buf, :, :, pl.ds(0, 128)],
                         out_hbm.at[s, :, w], wsems[buf])

    def wait_write(buf, s):
        pltpu.make_async_copy(outt_v.at[buf, :, :, pl.ds(0, 128)],
                              out_hbm.at[s, :, w], wsems[buf]).wait()

    lane_iota = lax.iota(jnp.int32, LANES)
    ftv = [(jnp.int32(c * 16) + lane_iota) >> 3 for c in range(D // LANES)]
    fsv = [(jnp.int32(c * 16) + lane_iota) & 7 for c in range(D // LANES)]

    def transpose_add(buf, s):
        pvec = [pos_v[s, pl.ds(c * LANES, LANES)] for c in range(D // LANES)]
        dst = outt_v.at[buf]

        @plsc.parallel_loop(0, 128, unroll=4)
        def _t(t, _pvec=pvec, _dst=dst):
            tv = jnp.full((LANES,), t, jnp.int32)
            for c in range(D // LANES):
                v = rows_v[buf, t, pl.ds(c * LANES, LANES)]
                plsc.store_scatter(_dst, [ftv[c], fsv[c], tv], v + _pvec[c])

    fire_gather(0, 0)
    fire_gather(1, 1)

    def body2(i, _):
        for buf in (0, 1):
            s = 2 * i + buf
            wait_gather(buf, s)

            @pl.when(i > 0)
            def _():
                wait_write(buf, s - 2)

            transpose_add(buf, s)
            fire_write(buf, s)

            @pl.when(s + 2 < S)
            def _():
                fire_gather(buf, s + 2)

        return 0

    lax.fori_loop(0, S // 2, body2, 0)
    wait_write(0, S - 2)
    wait_write(1, S - 1)


def kernel(inputs, token_table, pos_table):
    tablet = token_table.T
    tail = jnp.pad(token_table[NT_FULL * 128:], ((0, 0), (0, 128 - D)))
    scr = _prep(tablet, tail)
    idxt = inputs.astype(jnp.int32).T
    out = _emb(idxt, scr, pos_table)
    return out.transpose(2, 4, 0, 1, 3).reshape(B, S, D)


# speedup vs baseline: 1.6022x; 1.6022x over previous
import functools

import jax
import jax.numpy as jnp
from jax import lax
from jax.experimental import pallas as pl
from jax.experimental.pallas import tpu as pltpu
from jax.experimental.pallas import tpu_sc as plsc

B, S, D, V = 4096, 200, 64, 1000000
NC, NS = 2, 16
NW = NC * NS
BT = B // 128
FT = D // 8
LANES = 16
STRIDE = 129

NT_FULL = V // 128
TAIL = V - NT_FULL * 128
PER_W_BASE = NT_FULL // NW
PER_W_EXTRA = NT_FULL % NW

_mesh = plsc.VectorSubcoreMesh(core_axis_name="c", subcore_axis_name="s")


@functools.partial(
    pl.kernel,
    out_type=jax.ShapeDtypeStruct((V, 128), jnp.float32),
    mesh=_mesh,
    scratch_types=[
        pltpu.VMEM((2, D, 128), jnp.float32),
        pltpu.VMEM((2, 128, 128), jnp.float32),
        pltpu.VMEM((8 * 272,), jnp.float32),
        pltpu.SemaphoreType.DMA,
        pltpu.SemaphoreType.DMA,
        pltpu.SemaphoreType.DMA,
        pltpu.SemaphoreType.DMA,
    ],
    compiler_params=pltpu.CompilerParams(use_tc_tiling_on_sc=True,
                                         needs_layout_passes=False),
)
def _prep(tablet_hbm, tail_hbm, scr_hbm, in_v, st_v, blk_v, gsem0, gsem1, wsem0, wsem1):
    w = lax.axis_index("s") * NC + lax.axis_index("c")
    gsems = (gsem0, gsem1)
    wsems = (wsem0, wsem1)
    base = w * PER_W_BASE + jnp.minimum(w, PER_W_EXTRA)
    cnt = PER_W_BASE + jnp.where(w < PER_W_EXTRA, 1, 0)

    def rt_of(j):
        return pl.multiple_of((base + j) * 128, 128)

    def fire_read(buf, j):
        pltpu.async_copy(tablet_hbm.at[:, pl.ds(rt_of(j), 128)], in_v.at[buf],
                         gsems[buf])

    def wait_read(buf, j):
        pltpu.make_async_copy(tablet_hbm.at[:, pl.ds(rt_of(j), 128)],
                              in_v.at[buf], gsems[buf]).wait()

    def fire_write(buf, j):
        pltpu.async_copy(st_v.at[buf], scr_hbm.at[pl.ds(rt_of(j), 128)],
                         wsems[buf])

    def wait_write(buf, j):
        pltpu.make_async_copy(st_v.at[buf], scr_hbm.at[pl.ds(rt_of(j), 128)],
                              wsems[buf]).wait()

    lane_iota = lax.iota(jnp.int32, LANES)
    iota17 = lane_iota * 17

    def transpose(buf):
        @plsc.parallel_loop(0, 128 // LANES, unroll=2)
        def _t0(t0):
            blk = blk_v.at[pl.ds(t0 * 272, 272)]
            for f0 in range(D // LANES):
                for df in range(LANES):
                    v = in_v[buf, f0 * LANES + df, pl.ds(t0 * LANES, LANES)]
                    plsc.store_scatter(blk, [iota17 + df], v)
                for dt in range(LANES):
                    st_v[buf, t0 * LANES + dt, pl.ds(f0 * LANES, LANES)] = (
                        blk[pl.ds(dt * 17, LANES)])

    @pl.when(cnt > 0)
    def _():
        fire_read(0, 0)

    @pl.when(cnt > 1)
    def _():
        fire_read(1, 1)

    def body(j, _):
        for buf in (0, 1):
            jj = 2 * j + buf

            @pl.when(jj < cnt)
            def _():
                wait_read(buf, jj)

                @pl.when(jj >= 2)
                def _():
                    wait_write(buf, jj - 2)

                transpose(buf)
                fire_write(buf, jj)

                @pl.when(jj + 2 < cnt)
                def _():
                    fire_read(buf, jj + 2)

        return 0

    lax.fori_loop(0, (PER_W_BASE + 2) // 2, body, 0)

    @pl.when(cnt > 1)
    def _():
        wait_write(0, cnt - 2)

    @pl.when(cnt > 0)
    def _():
        wait_write(1, cnt - 1)

    @pl.when(w == NW - 1)
    def _():
        pltpu.sync_copy(tail_hbm, in_v.at[0, :, :])
        pltpu.sync_copy(in_v.at[0, :, :],
                        scr_hbm.at[pl.ds(NT_FULL * 128, TAIL)])


@functools.partial(
    pl.kernel,
    out_type=jax.ShapeDtypeStruct((S, FT, BT, 8, 128), jnp.float32),
    mesh=_mesh,
    scratch_types=[
        pltpu.VMEM((S, 128), jnp.int32),
        pltpu.VMEM((S, D), jnp.float32),
        pltpu.VMEM((2, 128, 128), jnp.float32),
        pltpu.VMEM((2, FT, 8, STRIDE), jnp.float32),
        pltpu.SemaphoreType.DMA,
        pltpu.SemaphoreType.DMA,
        pltpu.SemaphoreType.DMA,
        pltpu.SemaphoreType.DMA,
    ],
    compiler_params=pltpu.CompilerParams(use_tc_tiling_on_sc=False,
                                         needs_layout_passes=False),
)
def _emb(idxt_hbm, scr_hbm, pos_hbm, out_hbm,
         idx_v, pos_v, rows_v, outt_v, gsem0, gsem1, wsem0, wsem1):
    w = lax.axis_index("s") * NC + lax.axis_index("c")
    gsems = (gsem0, gsem1)
    wsems = (wsem0, wsem1)

    pltpu.sync_copy(idxt_hbm.at[:, pl.ds(pl.multiple_of(w * 128, 128), 128)], idx_v)
    pltpu.sync_copy(pos_hbm, pos_v)

    def fire_gather(buf, s):
        pltpu.async_copy(scr_hbm.at[idx_v.at[s]], rows_v.at[buf], gsems[buf])

    def wait_gather(buf, s):
        pltpu.make_async_copy(scr_hbm.at[idx_v.at[s]], rows_v.at[buf],
                              gsems[buf]).wait()

    def fire_write(buf, s):
        pltpu.async_copy(outt_v.at[buf, :, :, pl.ds(0, 128)],
                         out_hbm.at[s, :, w], wsems[buf])

    def wait_write(buf, s):
        pltpu.make_async_copy(outt_v.at[buf, :, :, pl.ds(0, 128)],
                              out_hbm.at[s, :, w], wsems[buf]).wait()

    lane_iota = lax.iota(jnp.int32, LANES)
    ftv = [(jnp.int32(c * 16) + lane_iota) >> 3 for c in range(D // LANES)]
    fsv = [(jnp.int32(c * 16) + lane_iota) & 7 for c in range(D // LANES)]

    def transpose_add(buf, s):
        pvec = [pos_v[s, pl.ds(c * LANES, LANES)] for c in range(D // LANES)]
        dst = outt_v.at[buf]

        @plsc.parallel_loop(0, 128, unroll=4)
        def _t(t, _pvec=pvec, _dst=dst):
            tv = jnp.full((LANES,), t, jnp.int32)
            for c in range(D // LANES):
                v = rows_v[buf, t, pl.ds(c * LANES, LANES)]
                plsc.store_scatter(_dst, [ftv[c], fsv[c], tv], v + _pvec[c])

    fire_gather(0, 0)
    fire_gather(1, 1)

    def body2(i, _):
        for buf in (0, 1):
            s = 2 * i + buf
            wait_gather(buf, s)

            @pl.when(i > 0)
            def _():
                wait_write(buf, s - 2)

            transpose_add(buf, s)
            fire_write(buf, s)

            @pl.when(s + 2 < S)
            def _():
                fire_gather(buf, s + 2)

        return 0

    lax.fori_loop(0, S // 2, body2, 0)
    wait_write(0, S - 2)
    wait_write(1, S - 1)


def kernel(inputs, token_table, pos_table):
    tablet = token_table.T
    tail = jnp.pad(token_table[NT_FULL * 128:], ((0, 0), (0, 128 - D)))
    scr = _prep(tablet, tail)
    idxt = inputs.astype(jnp.int32).T
    out = _emb(idxt, scr, pos_table)
    return out.transpose(2, 4, 0, 1, 3).reshape(B, S, D)


# speedup vs baseline: 1.6065x; 1.0027x over previous
import functools

import jax
import jax.numpy as jnp
from jax import lax
from jax.experimental import pallas as pl
from jax.experimental.pallas import tpu as pltpu
from jax.experimental.pallas import tpu_sc as plsc

B, S, D, V = 4096, 200, 64, 1000000
NC, NS = 2, 16
NW = NC * NS
BT = B // 128
FT = D // 8
LANES = 16
STRIDE = 129

NT_FULL = V // 128
TAIL = V - NT_FULL * 128
PER_W_BASE = NT_FULL // NW
PER_W_EXTRA = NT_FULL % NW

_mesh = plsc.VectorSubcoreMesh(core_axis_name="c", subcore_axis_name="s")


@functools.partial(
    pl.kernel,
    out_type=jax.ShapeDtypeStruct((V, 128), jnp.float32),
    mesh=_mesh,
    scratch_types=[
        pltpu.VMEM((2, D, 128), jnp.float32),
        pltpu.VMEM((2, 128, 128), jnp.float32),
        pltpu.VMEM((32 * 272,), jnp.float32),
        pltpu.SemaphoreType.DMA,
        pltpu.SemaphoreType.DMA,
        pltpu.SemaphoreType.DMA,
        pltpu.SemaphoreType.DMA,
    ],
    compiler_params=pltpu.CompilerParams(use_tc_tiling_on_sc=True,
                                         needs_layout_passes=False),
)
def _prep(tablet_hbm, tail_hbm, scr_hbm, in_v, st_v, blk_v, gsem0, gsem1, wsem0, wsem1):
    w = lax.axis_index("s") * NC + lax.axis_index("c")
    gsems = (gsem0, gsem1)
    wsems = (wsem0, wsem1)
    base = w * PER_W_BASE + jnp.minimum(w, PER_W_EXTRA)
    cnt = PER_W_BASE + jnp.where(w < PER_W_EXTRA, 1, 0)

    def rt_of(j):
        return pl.multiple_of((base + j) * 128, 128)

    def fire_read(buf, j):
        pltpu.async_copy(tablet_hbm.at[:, pl.ds(rt_of(j), 128)], in_v.at[buf],
                         gsems[buf])

    def wait_read(buf, j):
        pltpu.make_async_copy(tablet_hbm.at[:, pl.ds(rt_of(j), 128)],
                              in_v.at[buf], gsems[buf]).wait()

    def fire_write(buf, j):
        pltpu.async_copy(st_v.at[buf], scr_hbm.at[pl.ds(rt_of(j), 128)],
                         wsems[buf])

    def wait_write(buf, j):
        pltpu.make_async_copy(st_v.at[buf], scr_hbm.at[pl.ds(rt_of(j), 128)],
                              wsems[buf]).wait()

    lane_iota = lax.iota(jnp.int32, LANES)
    iota17 = lane_iota * 17

    def transpose(buf):
        @plsc.parallel_loop(0, 128 // LANES, unroll=2)
        def _t0(t0):
            for f0 in range(D // LANES):
                blk = blk_v.at[pl.ds((t0 * 4 + f0) * 272, 272)]
                for df in range(LANES):
                    v = in_v[buf, f0 * LANES + df, pl.ds(t0 * LANES, LANES)]
                    plsc.store_scatter(blk, [iota17 + df], v)
                for dt in range(LANES):
                    st_v[buf, t0 * LANES + dt, pl.ds(f0 * LANES, LANES)] = (
                        blk[pl.ds(dt * 17, LANES)])

    @pl.when(cnt > 0)
    def _():
        fire_read(0, 0)

    @pl.when(cnt > 1)
    def _():
        fire_read(1, 1)

    def body(j, _):
        for buf in (0, 1):
            jj = 2 * j + buf

            @pl.when(jj < cnt)
            def _():
                wait_read(buf, jj)

                @pl.when(jj >= 2)
                def _():
                    wait_write(buf, jj - 2)

                transpose(buf)
                fire_write(buf, jj)

                @pl.when(jj + 2 < cnt)
                def _():
                    fire_read(buf, jj + 2)

        return 0

    lax.fori_loop(0, (PER_W_BASE + 2) // 2, body, 0)

    @pl.when(cnt > 1)
    def _():
        wait_write(0, cnt - 2)

    @pl.when(cnt > 0)
    def _():
        wait_write(1, cnt - 1)

    @pl.when(w == NW - 1)
    def _():
        pltpu.sync_copy(tail_hbm, in_v.at[0, :, :])
        pltpu.sync_copy(in_v.at[0, :, :],
                        scr_hbm.at[pl.ds(NT_FULL * 128, TAIL)])


@functools.partial(
    pl.kernel,
    out_type=jax.ShapeDtypeStruct((S, FT, BT, 8, 128), jnp.float32),
    mesh=_mesh,
    scratch_types=[
        pltpu.VMEM((S, 128), jnp.int32),
        pltpu.VMEM((S, D), jnp.float32),
        pltpu.VMEM((2, 128, 128), jnp.float32),
        pltpu.VMEM((2, FT, 8, STRIDE), jnp.float32),
        pltpu.SemaphoreType.DMA,
        pltpu.SemaphoreType.DMA,
        pltpu.SemaphoreType.DMA,
        pltpu.SemaphoreType.DMA,
    ],
    compiler_params=pltpu.CompilerParams(use_tc_tiling_on_sc=False,
                                         needs_layout_passes=False),
)
def _emb(idxt_hbm, scr_hbm, pos_hbm, out_hbm,
         idx_v, pos_v, rows_v, outt_v, gsem0, gsem1, wsem0, wsem1):
    w = lax.axis_index("s") * NC + lax.axis_index("c")
    gsems = (gsem0, gsem1)
    wsems = (wsem0, wsem1)

    pltpu.sync_copy(idxt_hbm.at[:, pl.ds(pl.multiple_of(w * 128, 128), 128)], idx_v)
    pltpu.sync_copy(pos_hbm, pos_v)

    def fire_gather(buf, s):
        pltpu.async_copy(scr_hbm.at[idx_v.at[s]], rows_v.at[buf], gsems[buf])

    def wait_gather(buf, s):
        pltpu.make_async_copy(scr_hbm.at[idx_v.at[s]], rows_v.at[buf],
                              gsems[buf]).wait()

    def fire_write(buf, s):
        pltpu.async_copy(outt_v.at[buf, :, :, pl.ds(0, 128)],
                         out_hbm.at[s, :, w], wsems[buf])

    def wait_write(buf, s):
        pltpu.make_async_copy(outt_v.at[buf, :, :, pl.ds(0, 128)],
                              out_hbm.at[s, :, w], wsems[buf]).wait()

    lane_iota = lax.iota(jnp.int32, LANES)
    ftv = [(jnp.int32(c * 16) + lane_iota) >> 3 for c in range(D // LANES)]
    fsv = [(jnp.int32(c * 16) + lane_iota) & 7 for c in range(D // LANES)]

    def transpose_add(buf, s):
        pvec = [pos_v[s, pl.ds(c * LANES, LANES)] for c in range(D // LANES)]
        dst = outt_v.at[buf]

        @plsc.parallel_loop(0, 128, unroll=4)
        def _t(t, _pvec=pvec, _dst=dst):
            tv = jnp.full((LANES,), t, jnp.int32)
            for c in range(D // LANES):
                v = rows_v[buf, t, pl.ds(c * LANES, LANES)]
                plsc.store_scatter(_dst, [ftv[c], fsv[c], tv], v + _pvec[c])

    fire_gather(0, 0)
    fire_gather(1, 1)

    def body2(i, _):
        for buf in (0, 1):
            s = 2 * i + buf
            wait_gather(buf, s)

            @pl.when(i > 0)
            def _():
                wait_write(buf, s - 2)

            transpose_add(buf, s)
            fire_write(buf, s)

            @pl.when(s + 2 < S)
            def _():
                fire_gather(buf, s + 2)

        return 0

    lax.fori_loop(0, S // 2, body2, 0)
    wait_write(0, S - 2)
    wait_write(1, S - 1)


def kernel(inputs, token_table, pos_table):
    tablet = token_table.T
    tail = jnp.pad(token_table[NT_FULL * 128:], ((0, 0), (0, 128 - D)))
    scr = _prep(tablet, tail)
    idxt = inputs.astype(jnp.int32).T
    out = _emb(idxt, scr, pos_table)
    return out.transpose(2, 4, 0, 1, 3).reshape(B, S, D)
